# Initial kernel scaffold; baseline (speedup 1.0000x reference)
import jax
import jax.numpy as jnp
from jax.experimental import pallas as pl


def kernel(x, edge_attr, We0, be0, Wn0, bb0, g0, bt0, We1, be1, Wn1, bb1, g1, bt1, We2, be2, Wn2, bb2, g2, bt2, hg0, hbt0, hW1, hb1, hg1, hbt1, hW2, hb2, edge_index, batch):
    raise NotImplementedError("write your pallas kernel here")


# speedup vs baseline: 2.7520x; 2.7520x over previous
import functools

import jax
import jax.numpy as jnp
from jax import lax
from jax.experimental import pallas as pl
from jax.experimental.pallas import tpu as pltpu
from jax.experimental.pallas import tpu_sc as plsc

N = 10000
E = 320000
D = 128
G = 64
NPAD = 10240

NC = 2
NS = 16
NW = NC * NS
C = 80
EPW = E // NW
NCHUNK = EPW // C
SUPC = 25
NSUP = NCHUNK // SUPC
ZROWS = 40
NWB = N // ZROWS


def _edge_body(h_hbm, e_hbm, src_hbm, dst_hbm, out_hbm,
               src_v, dst_v, rows_v, e_v, zbuf, agg_sh, sem):
    c = lax.axis_index("c")
    s = lax.axis_index("s")
    wid = c * NS + s

    def zrow(r, carry):
        for dcol in range(D // 16):
            zbuf[r, pl.ds(dcol * 16, 16)] = jnp.zeros((16,), jnp.float32)
        return carry
    lax.fori_loop(0, ZROWS, zrow, None)
    for kk in range(-(-NWB // NS)):
        jwb = s + kk * NS

        @pl.when(jwb < NWB)
        def _(jwb=jwb):
            r0 = pl.multiple_of(jwb * ZROWS, 8)
            pltpu.sync_copy(zbuf, agg_sh.at[pl.ds(r0, ZROWS)])
    plsc.subcore_barrier()

    def sup(si, carry):
        pltpu.sync_copy(src_hbm.at[wid, si], src_v)
        pltpu.sync_copy(dst_hbm.at[wid, si], dst_v)

        def chunk(j, carry1):
            pltpu.async_copy(h_hbm.at[src_v.at[j]], rows_v, sem).wait()
            e0 = pl.multiple_of(wid * EPW + (si * SUPC + j) * C, 8)
            pltpu.sync_copy(e_hbm.at[pl.ds(e0, C)], e_v)

            def row(r, carry2):
                for dcol in range(D // 16):
                    sl = pl.ds(dcol * 16, 16)
                    rows_v[r, sl] = jnp.maximum(
                        rows_v[r, sl] + e_v[r, sl], 0.0)
                return carry2
            lax.fori_loop(0, C, row, None)
            pltpu.sync_copy(rows_v, agg_sh.at[dst_v.at[j]], add=True)
            return carry1
        lax.fori_loop(0, SUPC, chunk, None)
        return carry
    lax.fori_loop(0, NSUP, sup, None)

    plsc.subcore_barrier()
    for kk in range(-(-NWB // NS)):
        jwb = s + kk * NS

        @pl.when(jwb < NWB)
        def _(jwb=jwb):
            r0 = pl.multiple_of(jwb * ZROWS, 8)
            pltpu.sync_copy(agg_sh.at[pl.ds(r0, ZROWS)], zbuf)
            pltpu.sync_copy(zbuf, out_hbm.at[c, pl.ds(r0, ZROWS)])


@functools.cache
def _make_edge_kernel():
    return functools.partial(
        pl.kernel,
        mesh=plsc.VectorSubcoreMesh(core_axis_name="c",
                                    subcore_axis_name="s"),
        out_type=jax.ShapeDtypeStruct((NC, N, D), jnp.float32),
        scratch_types=[
            pltpu.VMEM((SUPC, C), jnp.int32),
            pltpu.VMEM((SUPC, C), jnp.int32),
            pltpu.VMEM((C, D), jnp.float32),
            pltpu.VMEM((C, D), jnp.float32),
            pltpu.VMEM((ZROWS, D), jnp.float32),
            pltpu.VMEM_SHARED((N, D), jnp.float32),
            pltpu.SemaphoreType.DMA,
        ],
    )(_edge_body)


def _edge_kernel(h, e, src2d, dst2d):
    return _make_edge_kernel()(h, e, src2d, dst2d)


_BE = 2000


def _e_matmul(ea, We, be):
    def body(ea_ref, We_ref, be_ref, o_ref):
        o_ref[...] = (
            jnp.dot(ea_ref[...], We_ref[...],
                    preferred_element_type=jnp.float32) + be_ref[...]
        )
    return pl.pallas_call(
        body,
        grid=(E // _BE,),
        in_specs=[
            pl.BlockSpec((_BE, 37), lambda i: (i, 0)),
            pl.BlockSpec((37, D), lambda i: (0, 0)),
            pl.BlockSpec((1, D), lambda i: (0, 0)),
        ],
        out_specs=pl.BlockSpec((_BE, D), lambda i: (i, 0)),
        out_shape=jax.ShapeDtypeStruct((E, D), jnp.float32),
    )(ea, We, be)


_BN_B = 2000


def _node_matmul(h, p0, p1, Wn, bb):
    def body(h_ref, p0_ref, p1_ref, Wn_ref, bb_ref, t_ref, s_ref, ss_ref):
        i = pl.program_id(0)
        t = jnp.dot(h_ref[...] + p0_ref[...] + p1_ref[...], Wn_ref[...],
                    preferred_element_type=jnp.float32) + bb_ref[...]
        t_ref[...] = t

        @pl.when(i == 0)
        def _():
            s_ref[...] = jnp.zeros_like(s_ref)
            ss_ref[...] = jnp.zeros_like(ss_ref)
        s_ref[...] += jnp.sum(t, axis=0, keepdims=True)
        ss_ref[...] += jnp.sum(t * t, axis=0, keepdims=True)

    return pl.pallas_call(
        body,
        grid=(N // _BN_B,),
        in_specs=[
            pl.BlockSpec((_BN_B, D), lambda i: (i, 0)),
            pl.BlockSpec((_BN_B, D), lambda i: (i, 0)),
            pl.BlockSpec((_BN_B, D), lambda i: (i, 0)),
            pl.BlockSpec((D, D), lambda i: (0, 0)),
            pl.BlockSpec((1, D), lambda i: (0, 0)),
        ],
        out_specs=[
            pl.BlockSpec((_BN_B, D), lambda i: (i, 0)),
            pl.BlockSpec((1, D), lambda i: (0, 0)),
            pl.BlockSpec((1, D), lambda i: (0, 0)),
        ],
        out_shape=[
            jax.ShapeDtypeStruct((N, D), jnp.float32),
            jax.ShapeDtypeStruct((1, D), jnp.float32),
            jax.ShapeDtypeStruct((1, D), jnp.float32),
        ],
    )(h, p0, p1, Wn, bb)


def _bn_relu(t, ssum, ssq, g, bt):
    def body(t_ref, s_ref, ss_ref, g_ref, bt_ref, o_ref):
        mu = s_ref[...] / N
        var = ss_ref[...] / N - mu * mu
        o_ref[...] = jnp.maximum(
            (t_ref[...] - mu) * lax.rsqrt(var + 1e-5) * g_ref[...]
            + bt_ref[...], 0.0)

    return pl.pallas_call(
        body,
        grid=(N // _BN_B,),
        in_specs=[
            pl.BlockSpec((_BN_B, D), lambda i: (i, 0)),
            pl.BlockSpec((1, D), lambda i: (0, 0)),
            pl.BlockSpec((1, D), lambda i: (0, 0)),
            pl.BlockSpec((1, D), lambda i: (0, 0)),
            pl.BlockSpec((1, D), lambda i: (0, 0)),
        ],
        out_specs=pl.BlockSpec((_BN_B, D), lambda i: (i, 0)),
        out_shape=jax.ShapeDtypeStruct((N, D), jnp.float32),
    )(t, ssum, ssq, g, bt)


def _head(hp, bp, hg0, hbt0, hW1, hb1, hg1, hbt1, w2, b2):
    def bn(v, gg, bb_):
        mu = jnp.mean(v, axis=0, keepdims=True)
        var = jnp.mean((v - mu) ** 2, axis=0, keepdims=True)
        return (v - mu) * lax.rsqrt(var + 1e-5) * gg + bb_

    def body(h_ref, b_ref, hg0_r, hbt0_r, hW1_r, hb1_r, hg1_r, hbt1_r,
             w2_r, b2_r, o_ref):
        gid = lax.broadcasted_iota(jnp.int32, (G, NPAD), 0)
        m = (gid == b_ref[...]).astype(jnp.float32)
        sums = jnp.dot(m, h_ref[...], preferred_element_type=jnp.float32)
        cnt = jnp.sum(m, axis=1, keepdims=True)
        gp = sums / jnp.maximum(cnt, 1.0)
        o = bn(gp, hg0_r[...], hbt0_r[...])
        o = jnp.maximum(
            jnp.dot(o, hW1_r[...], preferred_element_type=jnp.float32)
            + hb1_r[...], 0.0)
        o = bn(o, hg1_r[...], hbt1_r[...])
        o_ref[...] = jnp.sum(o * w2_r[...], axis=1, keepdims=True) + b2_r[...]

    return pl.pallas_call(
        body,
        out_shape=jax.ShapeDtypeStruct((G, 1), jnp.float32),
    )(hp, bp, hg0, hbt0, hW1, hb1, hg1, hbt1, w2, b2)


def kernel(x, edge_attr, We0, be0, Wn0, bb0, g0, bt0, We1, be1, Wn1, bb1,
           g1, bt1, We2, be2, Wn2, bb2, g2, bt2, hg0, hbt0, hW1, hb1, hg1,
           hbt1, hW2, hb2, edge_index, batch):
    x0 = jax.nn.one_hot(x[:, 0].astype(jnp.int32), 119, dtype=jnp.float32)
    h = jnp.concatenate([x0, x[:, 1:]], axis=1)
    ea0 = jax.nn.one_hot(edge_attr[:, 0].astype(jnp.int32), 22,
                         dtype=jnp.float32)
    ea = jnp.concatenate([ea0, edge_attr[:, 1:]], axis=1)
    src2d = edge_index[0].reshape(NW, NSUP, SUPC, C)
    dst2d = edge_index[1].reshape(NW, NSUP, SUPC, C)

    layers = [(We0, be0, Wn0, bb0, g0, bt0),
              (We1, be1, Wn1, bb1, g1, bt1),
              (We2, be2, Wn2, bb2, g2, bt2)]
    for We, be, Wn, bb, g, bt in layers:
        e = _e_matmul(ea, We, be.reshape(1, D))
        parts = _edge_kernel(h, e, src2d, dst2d)
        t, ssum, ssq = _node_matmul(h, parts[0], parts[1], Wn,
                                    bb.reshape(1, D))
        h = _bn_relu(t, ssum, ssq, g.reshape(1, D), bt.reshape(1, D))

    hp = jnp.pad(h, ((0, NPAD - N), (0, 0)))
    bp = jnp.pad(batch, (0, NPAD - N), constant_values=G).reshape(1, NPAD)
    return _head(hp, bp, hg0.reshape(1, D), hbt0.reshape(1, D), hW1,
                 hb1.reshape(1, D), hg1.reshape(1, D), hbt1.reshape(1, D),
                 hW2.reshape(1, D), hb2.reshape(1, 1))


# speedup vs baseline: 3.2957x; 1.1976x over previous
import functools

import jax
import jax.numpy as jnp
from jax import lax
from jax.experimental import pallas as pl
from jax.experimental.pallas import tpu as pltpu
from jax.experimental.pallas import tpu_sc as plsc

N = 10000
E = 320000
D = 128
G = 64
NPAD = 10240

NC = 2
NS = 16
NW = NC * NS
C = 80
EPW = E // NW
NCHUNK = EPW // C
SUPC = 25
NSUP = NCHUNK // SUPC
ZROWS = 80
NWB = N // ZROWS


def _edge_body(h_hbm, e_hbm, src_hbm, dst_hbm, out_hbm,
               src_v, dst_v, rows0, rows1, e_v, agg_sh,
               gsem0, gsem1, ssem0, ssem1):
    c = lax.axis_index("c")
    s = lax.axis_index("s")
    wid = c * NS + s

    def zrow(r, carry):
        for dcol in range(D // 16):
            rows0[r, pl.ds(dcol * 16, 16)] = jnp.zeros((16,), jnp.float32)
        return carry
    lax.fori_loop(0, C, zrow, None)
    for kk in range(-(-NWB // NS)):
        jwb = s + kk * NS

        @pl.when(jwb < NWB)
        def _(jwb=jwb):
            r0 = pl.multiple_of(jwb * ZROWS, 8)
            pltpu.sync_copy(rows0, agg_sh.at[pl.ds(r0, ZROWS)])
    plsc.subcore_barrier()

    def compute_relu(rows_a):
        def row(r, carry2):
            for dcol in range(D // 16):
                sl = pl.ds(dcol * 16, 16)
                rows_a[r, sl] = jnp.maximum(rows_a[r, sl] + e_v[r, sl], 0.0)
            return carry2
        lax.fori_loop(0, C, row, None)

    def sup(si, carry):
        pltpu.sync_copy(src_hbm.at[wid, si], src_v)
        pltpu.sync_copy(dst_hbm.at[wid, si], dst_v)
        pltpu.async_copy(h_hbm.at[src_v.at[0]], rows0, gsem0)

        def step(j, rows_a, gsem_a, ssem_a, rows_b, gsem_b, ssem_b):
            @pl.when(j >= 1)
            def _():
                pltpu.make_async_copy(
                    rows_b, agg_sh.at[dst_v.at[j - 1]], ssem_b).wait()

            @pl.when(j + 1 < SUPC)
            def _():
                pltpu.async_copy(h_hbm.at[src_v.at[j + 1]], rows_b, gsem_b)
            pltpu.make_async_copy(h_hbm.at[src_v.at[j]], rows_a,
                                  gsem_a).wait()
            e0 = pl.multiple_of(wid * EPW + (si * SUPC + j) * C, 8)
            pltpu.sync_copy(e_hbm.at[pl.ds(e0, C)], e_v)
            compute_relu(rows_a)
            pltpu.async_copy(rows_a, agg_sh.at[dst_v.at[j]], ssem_a,
                             add=True)

        def chunk(j, carry1):
            @pl.when(j % 2 == 0)
            def _():
                step(j, rows0, gsem0, ssem0, rows1, gsem1, ssem1)

            @pl.when(j % 2 == 1)
            def _():
                step(j, rows1, gsem1, ssem1, rows0, gsem0, ssem0)
            return carry1
        lax.fori_loop(0, SUPC, chunk, None)
        lastp = (SUPC - 1) % 2
        pltpu.make_async_copy(
            rows0 if lastp == 0 else rows1,
            agg_sh.at[dst_v.at[SUPC - 1]],
            ssem0 if lastp == 0 else ssem1).wait()
        return carry
    lax.fori_loop(0, NSUP, sup, None)

    plsc.subcore_barrier()
    for kk in range(-(-NWB // NS)):
        jwb = s + kk * NS

        @pl.when(jwb < NWB)
        def _(jwb=jwb):
            r0 = pl.multiple_of(jwb * ZROWS, 8)
            pltpu.sync_copy(agg_sh.at[pl.ds(r0, ZROWS)], rows0)
            pltpu.sync_copy(rows0, out_hbm.at[c, pl.ds(r0, ZROWS)])


@functools.cache
def _make_edge_kernel():
    return functools.partial(
        pl.kernel,
        mesh=plsc.VectorSubcoreMesh(core_axis_name="c",
                                    subcore_axis_name="s"),
        out_type=jax.ShapeDtypeStruct((NC, N, D), jnp.float32),
        scratch_types=[
            pltpu.VMEM((SUPC, C), jnp.int32),
            pltpu.VMEM((SUPC, C), jnp.int32),
            pltpu.VMEM((C, D), jnp.float32),
            pltpu.VMEM((C, D), jnp.float32),
            pltpu.VMEM((C, D), jnp.float32),
            pltpu.VMEM_SHARED((N, D), jnp.float32),
            pltpu.SemaphoreType.DMA,
            pltpu.SemaphoreType.DMA,
            pltpu.SemaphoreType.DMA,
            pltpu.SemaphoreType.DMA,
        ],
    )(_edge_body)


def _edge_kernel(h, e, src2d, dst2d):
    return _make_edge_kernel()(h, e, src2d, dst2d)


_BE = 2000


def _e_matmul(ea, We, be):
    def body(ea_ref, We_ref, be_ref, o_ref):
        o_ref[...] = (
            jnp.dot(ea_ref[...], We_ref[...],
                    preferred_element_type=jnp.float32) + be_ref[...]
        )
    return pl.pallas_call(
        body,
        grid=(E // _BE,),
        in_specs=[
            pl.BlockSpec((_BE, 37), lambda i: (i, 0)),
            pl.BlockSpec((37, D), lambda i: (0, 0)),
            pl.BlockSpec((1, D), lambda i: (0, 0)),
        ],
        out_specs=pl.BlockSpec((_BE, D), lambda i: (i, 0)),
        out_shape=jax.ShapeDtypeStruct((E, D), jnp.float32),
    )(ea, We, be)


_BN_B = 2000


def _node_matmul(h, p0, p1, Wn, bb):
    def body(h_ref, p0_ref, p1_ref, Wn_ref, bb_ref, t_ref, s_ref, ss_ref):
        i = pl.program_id(0)
        t = jnp.dot(h_ref[...] + p0_ref[...] + p1_ref[...], Wn_ref[...],
                    preferred_element_type=jnp.float32) + bb_ref[...]
        t_ref[...] = t

        @pl.when(i == 0)
        def _():
            s_ref[...] = jnp.zeros_like(s_ref)
            ss_ref[...] = jnp.zeros_like(ss_ref)
        s_ref[...] += jnp.sum(t, axis=0, keepdims=True)
        ss_ref[...] += jnp.sum(t * t, axis=0, keepdims=True)

    return pl.pallas_call(
        body,
        grid=(N // _BN_B,),
        in_specs=[
            pl.BlockSpec((_BN_B, D), lambda i: (i, 0)),
            pl.BlockSpec((_BN_B, D), lambda i: (i, 0)),
            pl.BlockSpec((_BN_B, D), lambda i: (i, 0)),
            pl.BlockSpec((D, D), lambda i: (0, 0)),
            pl.BlockSpec((1, D), lambda i: (0, 0)),
        ],
        out_specs=[
            pl.BlockSpec((_BN_B, D), lambda i: (i, 0)),
            pl.BlockSpec((1, D), lambda i: (0, 0)),
            pl.BlockSpec((1, D), lambda i: (0, 0)),
        ],
        out_shape=[
            jax.ShapeDtypeStruct((N, D), jnp.float32),
            jax.ShapeDtypeStruct((1, D), jnp.float32),
            jax.ShapeDtypeStruct((1, D), jnp.float32),
        ],
    )(h, p0, p1, Wn, bb)


def _bn_relu(t, ssum, ssq, g, bt):
    def body(t_ref, s_ref, ss_ref, g_ref, bt_ref, o_ref):
        mu = s_ref[...] / N
        var = ss_ref[...] / N - mu * mu
        o_ref[...] = jnp.maximum(
            (t_ref[...] - mu) * lax.rsqrt(var + 1e-5) * g_ref[...]
            + bt_ref[...], 0.0)

    return pl.pallas_call(
        body,
        grid=(N // _BN_B,),
        in_specs=[
            pl.BlockSpec((_BN_B, D), lambda i: (i, 0)),
            pl.BlockSpec((1, D), lambda i: (0, 0)),
            pl.BlockSpec((1, D), lambda i: (0, 0)),
            pl.BlockSpec((1, D), lambda i: (0, 0)),
            pl.BlockSpec((1, D), lambda i: (0, 0)),
        ],
        out_specs=pl.BlockSpec((_BN_B, D), lambda i: (i, 0)),
        out_shape=jax.ShapeDtypeStruct((N, D), jnp.float32),
    )(t, ssum, ssq, g, bt)


def _head(hp, bp, hg0, hbt0, hW1, hb1, hg1, hbt1, w2, b2):
    def bn(v, gg, bb_):
        mu = jnp.mean(v, axis=0, keepdims=True)
        var = jnp.mean((v - mu) ** 2, axis=0, keepdims=True)
        return (v - mu) * lax.rsqrt(var + 1e-5) * gg + bb_

    def body(h_ref, b_ref, hg0_r, hbt0_r, hW1_r, hb1_r, hg1_r, hbt1_r,
             w2_r, b2_r, o_ref):
        gid = lax.broadcasted_iota(jnp.int32, (G, NPAD), 0)
        m = (gid == b_ref[...]).astype(jnp.float32)
        sums = jnp.dot(m, h_ref[...], preferred_element_type=jnp.float32)
        cnt = jnp.sum(m, axis=1, keepdims=True)
        gp = sums / jnp.maximum(cnt, 1.0)
        o = bn(gp, hg0_r[...], hbt0_r[...])
        o = jnp.maximum(
            jnp.dot(o, hW1_r[...], preferred_element_type=jnp.float32)
            + hb1_r[...], 0.0)
        o = bn(o, hg1_r[...], hbt1_r[...])
        o_ref[...] = jnp.sum(o * w2_r[...], axis=1, keepdims=True) + b2_r[...]

    return pl.pallas_call(
        body,
        out_shape=jax.ShapeDtypeStruct((G, 1), jnp.float32),
    )(hp, bp, hg0, hbt0, hW1, hb1, hg1, hbt1, w2, b2)


def kernel(x, edge_attr, We0, be0, Wn0, bb0, g0, bt0, We1, be1, Wn1, bb1,
           g1, bt1, We2, be2, Wn2, bb2, g2, bt2, hg0, hbt0, hW1, hb1, hg1,
           hbt1, hW2, hb2, edge_index, batch):
    x0 = jax.nn.one_hot(x[:, 0].astype(jnp.int32), 119, dtype=jnp.float32)
    h = jnp.concatenate([x0, x[:, 1:]], axis=1)
    ea0 = jax.nn.one_hot(edge_attr[:, 0].astype(jnp.int32), 22,
                         dtype=jnp.float32)
    ea = jnp.concatenate([ea0, edge_attr[:, 1:]], axis=1)
    src2d = edge_index[0].reshape(NW, NSUP, SUPC, C)
    dst2d = edge_index[1].reshape(NW, NSUP, SUPC, C)

    layers = [(We0, be0, Wn0, bb0, g0, bt0),
              (We1, be1, Wn1, bb1, g1, bt1),
              (We2, be2, Wn2, bb2, g2, bt2)]
    es = [_e_matmul(ea, We, be.reshape(1, D)) for We, be, *_ in layers]
    for e, (We, be, Wn, bb, g, bt) in zip(es, layers):
        parts = _edge_kernel(h, e, src2d, dst2d)
        t, ssum, ssq = _node_matmul(h, parts[0], parts[1], Wn,
                                    bb.reshape(1, D))
        h = _bn_relu(t, ssum, ssq, g.reshape(1, D), bt.reshape(1, D))

    hp = jnp.pad(h, ((0, NPAD - N), (0, 0)))
    bp = jnp.pad(batch, (0, NPAD - N), constant_values=G).reshape(1, NPAD)
    return _head(hp, bp, hg0.reshape(1, D), hbt0.reshape(1, D), hW1,
                 hb1.reshape(1, D), hg1.reshape(1, D), hbt1.reshape(1, D),
                 hW2.reshape(1, D), hb2.reshape(1, 1))


# speedup vs baseline: 4.3603x; 1.3230x over previous
import functools

import jax
import jax.numpy as jnp
from jax import lax
from jax.experimental import pallas as pl
from jax.experimental.pallas import tpu as pltpu
from jax.experimental.pallas import tpu_sc as plsc

N = 10000
E = 320000
D = 128
G = 64
NPAD = 10240

NC = 2
NS = 16
NW = NC * NS
C = 80
EPW = E // NW
NCHUNK = EPW // C
SUPC = 25
NSUP = NCHUNK // SUPC
ZROWS = 80
NWB = N // ZROWS


def _edge_body(h_hbm, e_hbm, src_hbm, dst_hbm, out_hbm,
               src_v, dst_v, rows0, rows1, e_v0, e_v1, agg_sh,
               gsem0, gsem1, ssem0, ssem1, esem0, esem1):
    c = lax.axis_index("c")
    s = lax.axis_index("s")
    wid = c * NS + s

    def zrow(r, carry):
        for dcol in range(D // 16):
            rows0[r, pl.ds(dcol * 16, 16)] = jnp.zeros((16,), jnp.float32)
        return carry
    lax.fori_loop(0, C, zrow, None)
    for kk in range(-(-NWB // NS)):
        jwb = s + kk * NS

        @pl.when(jwb < NWB)
        def _(jwb=jwb):
            r0 = pl.multiple_of(jwb * ZROWS, 8)
            pltpu.sync_copy(rows0, agg_sh.at[pl.ds(r0, ZROWS)])
    plsc.subcore_barrier()

    def compute_relu(rows_a, e_a):
        def row(r4, carry2):
            for u in range(4):
                r = r4 * 4 + u
                for dcol in range(D // 16):
                    sl = pl.ds(dcol * 16, 16)
                    rows_a[r, sl] = jnp.maximum(
                        rows_a[r, sl] + e_a[r, sl], 0.0)
            return carry2
        lax.fori_loop(0, C // 4, row, None)

    def e_off(si, j):
        return pl.multiple_of(wid * EPW + (si * SUPC + j) * C, 8)

    def sup(si, carry):
        pltpu.sync_copy(src_hbm.at[wid, si], src_v)
        pltpu.sync_copy(dst_hbm.at[wid, si], dst_v)
        pltpu.async_copy(h_hbm.at[src_v.at[0]], rows0, gsem0)
        pltpu.async_copy(e_hbm.at[pl.ds(e_off(si, 0), C)], e_v0, esem0)

        def step(j, rows_a, e_a, gsem_a, ssem_a, esem_a,
                 rows_b, e_b, gsem_b, ssem_b, esem_b):
            @pl.when(j >= 1)
            def _():
                pltpu.make_async_copy(
                    rows_b, agg_sh.at[dst_v.at[j - 1]], ssem_b).wait()

            @pl.when(j + 1 < SUPC)
            def _():
                pltpu.async_copy(h_hbm.at[src_v.at[j + 1]], rows_b, gsem_b)
                pltpu.async_copy(e_hbm.at[pl.ds(e_off(si, j + 1), C)],
                                 e_b, esem_b)
            pltpu.make_async_copy(h_hbm.at[src_v.at[j]], rows_a,
                                  gsem_a).wait()
            pltpu.make_async_copy(e_hbm.at[pl.ds(e_off(si, j), C)], e_a,
                                  esem_a).wait()
            compute_relu(rows_a, e_a)
            pltpu.async_copy(rows_a, agg_sh.at[dst_v.at[j]], ssem_a,
                             add=True)

        def chunk(j, carry1):
            @pl.when(j % 2 == 0)
            def _():
                step(j, rows0, e_v0, gsem0, ssem0, esem0,
                     rows1, e_v1, gsem1, ssem1, esem1)

            @pl.when(j % 2 == 1)
            def _():
                step(j, rows1, e_v1, gsem1, ssem1, esem1,
                     rows0, e_v0, gsem0, ssem0, esem0)
            return carry1
        lax.fori_loop(0, SUPC, chunk, None)
        lastp = (SUPC - 1) % 2
        pltpu.make_async_copy(
            rows0 if lastp == 0 else rows1,
            agg_sh.at[dst_v.at[SUPC - 1]],
            ssem0 if lastp == 0 else ssem1).wait()
        return carry
    lax.fori_loop(0, NSUP, sup, None)

    plsc.subcore_barrier()
    for kk in range(-(-NWB // NS)):
        jwb = s + kk * NS

        @pl.when(jwb < NWB)
        def _(jwb=jwb):
            r0 = pl.multiple_of(jwb * ZROWS, 8)
            pltpu.sync_copy(agg_sh.at[pl.ds(r0, ZROWS)], rows0)
            pltpu.sync_copy(rows0, out_hbm.at[c, pl.ds(r0, ZROWS)])


@functools.cache
def _make_edge_kernel():
    return functools.partial(
        pl.kernel,
        mesh=plsc.VectorSubcoreMesh(core_axis_name="c",
                                    subcore_axis_name="s"),
        out_type=jax.ShapeDtypeStruct((NC, N, D), jnp.float32),
        scratch_types=[
            pltpu.VMEM((SUPC, C), jnp.int32),
            pltpu.VMEM((SUPC, C), jnp.int32),
            pltpu.VMEM((C, D), jnp.float32),
            pltpu.VMEM((C, D), jnp.float32),
            pltpu.VMEM((C, D), jnp.float32),
            pltpu.VMEM((C, D), jnp.float32),
            pltpu.VMEM_SHARED((N, D), jnp.float32),
            pltpu.SemaphoreType.DMA,
            pltpu.SemaphoreType.DMA,
            pltpu.SemaphoreType.DMA,
            pltpu.SemaphoreType.DMA,
            pltpu.SemaphoreType.DMA,
            pltpu.SemaphoreType.DMA,
        ],
    )(_edge_body)


def _edge_kernel(h, e, src2d, dst2d):
    return _make_edge_kernel()(h, e, src2d, dst2d)


_BE = 2000


def _e_matmul(ea, We, be):
    def body(ea_ref, We_ref, be_ref, o_ref):
        o_ref[...] = (
            jnp.dot(ea_ref[...], We_ref[...],
                    preferred_element_type=jnp.float32) + be_ref[...]
        )
    return pl.pallas_call(
        body,
        grid=(E // _BE,),
        in_specs=[
            pl.BlockSpec((_BE, 37), lambda i: (i, 0)),
            pl.BlockSpec((37, D), lambda i: (0, 0)),
            pl.BlockSpec((1, D), lambda i: (0, 0)),
        ],
        out_specs=pl.BlockSpec((_BE, D), lambda i: (i, 0)),
        out_shape=jax.ShapeDtypeStruct((E, D), jnp.float32),
    )(ea, We, be)


_BN_B = 2000


def _node_matmul(h, p0, p1, Wn, bb):
    def body(h_ref, p0_ref, p1_ref, Wn_ref, bb_ref, t_ref, s_ref, ss_ref):
        i = pl.program_id(0)
        t = jnp.dot(h_ref[...] + p0_ref[...] + p1_ref[...], Wn_ref[...],
                    preferred_element_type=jnp.float32) + bb_ref[...]
        t_ref[...] = t

        @pl.when(i == 0)
        def _():
            s_ref[...] = jnp.zeros_like(s_ref)
            ss_ref[...] = jnp.zeros_like(ss_ref)
        s_ref[...] += jnp.sum(t, axis=0, keepdims=True)
        ss_ref[...] += jnp.sum(t * t, axis=0, keepdims=True)

    return pl.pallas_call(
        body,
        grid=(N // _BN_B,),
        in_specs=[
            pl.BlockSpec((_BN_B, D), lambda i: (i, 0)),
            pl.BlockSpec((_BN_B, D), lambda i: (i, 0)),
            pl.BlockSpec((_BN_B, D), lambda i: (i, 0)),
            pl.BlockSpec((D, D), lambda i: (0, 0)),
            pl.BlockSpec((1, D), lambda i: (0, 0)),
        ],
        out_specs=[
            pl.BlockSpec((_BN_B, D), lambda i: (i, 0)),
            pl.BlockSpec((1, D), lambda i: (0, 0)),
            pl.BlockSpec((1, D), lambda i: (0, 0)),
        ],
        out_shape=[
            jax.ShapeDtypeStruct((N, D), jnp.float32),
            jax.ShapeDtypeStruct((1, D), jnp.float32),
            jax.ShapeDtypeStruct((1, D), jnp.float32),
        ],
    )(h, p0, p1, Wn, bb)


def _bn_relu(t, ssum, ssq, g, bt):
    def body(t_ref, s_ref, ss_ref, g_ref, bt_ref, o_ref):
        mu = s_ref[...] / N
        var = ss_ref[...] / N - mu * mu
        o_ref[...] = jnp.maximum(
            (t_ref[...] - mu) * lax.rsqrt(var + 1e-5) * g_ref[...]
            + bt_ref[...], 0.0)

    return pl.pallas_call(
        body,
        grid=(N // _BN_B,),
        in_specs=[
            pl.BlockSpec((_BN_B, D), lambda i: (i, 0)),
            pl.BlockSpec((1, D), lambda i: (0, 0)),
            pl.BlockSpec((1, D), lambda i: (0, 0)),
            pl.BlockSpec((1, D), lambda i: (0, 0)),
            pl.BlockSpec((1, D), lambda i: (0, 0)),
        ],
        out_specs=pl.BlockSpec((_BN_B, D), lambda i: (i, 0)),
        out_shape=jax.ShapeDtypeStruct((N, D), jnp.float32),
    )(t, ssum, ssq, g, bt)


def _head(hp, bp, hg0, hbt0, hW1, hb1, hg1, hbt1, w2, b2):
    def bn(v, gg, bb_):
        mu = jnp.mean(v, axis=0, keepdims=True)
        var = jnp.mean((v - mu) ** 2, axis=0, keepdims=True)
        return (v - mu) * lax.rsqrt(var + 1e-5) * gg + bb_

    def body(h_ref, b_ref, hg0_r, hbt0_r, hW1_r, hb1_r, hg1_r, hbt1_r,
             w2_r, b2_r, o_ref):
        gid = lax.broadcasted_iota(jnp.int32, (G, NPAD), 0)
        m = (gid == b_ref[...]).astype(jnp.float32)
        sums = jnp.dot(m, h_ref[...], preferred_element_type=jnp.float32)
        cnt = jnp.sum(m, axis=1, keepdims=True)
        gp = sums / jnp.maximum(cnt, 1.0)
        o = bn(gp, hg0_r[...], hbt0_r[...])
        o = jnp.maximum(
            jnp.dot(o, hW1_r[...], preferred_element_type=jnp.float32)
            + hb1_r[...], 0.0)
        o = bn(o, hg1_r[...], hbt1_r[...])
        o_ref[...] = jnp.sum(o * w2_r[...], axis=1, keepdims=True) + b2_r[...]

    return pl.pallas_call(
        body,
        out_shape=jax.ShapeDtypeStruct((G, 1), jnp.float32),
    )(hp, bp, hg0, hbt0, hW1, hb1, hg1, hbt1, w2, b2)


def kernel(x, edge_attr, We0, be0, Wn0, bb0, g0, bt0, We1, be1, Wn1, bb1,
           g1, bt1, We2, be2, Wn2, bb2, g2, bt2, hg0, hbt0, hW1, hb1, hg1,
           hbt1, hW2, hb2, edge_index, batch):
    x0 = jax.nn.one_hot(x[:, 0].astype(jnp.int32), 119, dtype=jnp.float32)
    h = jnp.concatenate([x0, x[:, 1:]], axis=1)
    ea0 = jax.nn.one_hot(edge_attr[:, 0].astype(jnp.int32), 22,
                         dtype=jnp.float32)
    ea = jnp.concatenate([ea0, edge_attr[:, 1:]], axis=1)
    src2d = edge_index[0].reshape(NW, NSUP, SUPC, C)
    dst2d = edge_index[1].reshape(NW, NSUP, SUPC, C)

    layers = [(We0, be0, Wn0, bb0, g0, bt0),
              (We1, be1, Wn1, bb1, g1, bt1),
              (We2, be2, Wn2, bb2, g2, bt2)]
    es = [_e_matmul(ea, We, be.reshape(1, D)) for We, be, *_ in layers]
    for e, (We, be, Wn, bb, g, bt) in zip(es, layers):
        parts = _edge_kernel(h, e, src2d, dst2d)
        t, ssum, ssq = _node_matmul(h, parts[0], parts[1], Wn,
                                    bb.reshape(1, D))
        h = _bn_relu(t, ssum, ssq, g.reshape(1, D), bt.reshape(1, D))

    hp = jnp.pad(h, ((0, NPAD - N), (0, 0)))
    bp = jnp.pad(batch, (0, NPAD - N), constant_values=G).reshape(1, NPAD)
    return _head(hp, bp, hg0.reshape(1, D), hbt0.reshape(1, D), hW1,
                 hb1.reshape(1, D), hg1.reshape(1, D), hbt1.reshape(1, D),
                 hW2.reshape(1, D), hb2.reshape(1, 1))


# speedup vs baseline: 4.3705x; 1.0023x over previous
import functools

import jax
import jax.numpy as jnp
from jax import lax
from jax.experimental import pallas as pl
from jax.experimental.pallas import tpu as pltpu
from jax.experimental.pallas import tpu_sc as plsc

N = 10000
E = 320000
D = 128
G = 64
NPAD = 10240

NC = 2
NS = 16
NW = NC * NS
C = 80
EPW = E // NW
NCHUNK = EPW // C
SUPC = 25
NSUP = NCHUNK // SUPC
ZROWS = 80
NWB = N // ZROWS


def _edge_body(h_hbm, e_hbm, src_hbm, dst_hbm, out_hbm,
               src_v, dst_v, rows0, rows1, e_v0, e_v1, agg_sh,
               gsem0, gsem1, ssem0, ssem1, esem0, esem1):
    c = lax.axis_index("c")
    s = lax.axis_index("s")
    wid = c * NS + s

    def zrow(r, carry):
        for dcol in range(D // 16):
            rows0[r, pl.ds(dcol * 16, 16)] = jnp.zeros((16,), jnp.float32)
        return carry
    lax.fori_loop(0, C, zrow, None)
    for kk in range(-(-NWB // NS)):
        jwb = s + kk * NS

        @pl.when(jwb < NWB)
        def _(jwb=jwb):
            r0 = pl.multiple_of(jwb * ZROWS, 8)
            pltpu.sync_copy(rows0, agg_sh.at[pl.ds(r0, ZROWS)])
    plsc.subcore_barrier()

    def compute_relu(rows_a, e_a):
        def row(r4, carry2):
            for u in range(4):
                r = r4 * 4 + u
                for dcol in range(D // 16):
                    sl = pl.ds(dcol * 16, 16)
                    rows_a[r, sl] = jnp.maximum(
                        rows_a[r, sl] + e_a[r, sl], 0.0)
            return carry2
        lax.fori_loop(0, C // 4, row, None)

    def e_off(si, j):
        return pl.multiple_of(wid * EPW + (si * SUPC + j) * C, 8)

    def sup(si, carry):
        pltpu.sync_copy(src_hbm.at[wid, si], src_v)
        pltpu.sync_copy(dst_hbm.at[wid, si], dst_v)
        pltpu.async_copy(h_hbm.at[src_v.at[0]], rows0, gsem0)
        pltpu.async_copy(e_hbm.at[pl.ds(e_off(si, 0), C)], e_v0, esem0)

        def step(j, rows_a, e_a, gsem_a, ssem_a, esem_a,
                 rows_b, e_b, gsem_b, ssem_b, esem_b):
            @pl.when(j >= 1)
            def _():
                pltpu.make_async_copy(
                    rows_b, agg_sh.at[dst_v.at[j - 1]], ssem_b).wait()

            @pl.when(j + 1 < SUPC)
            def _():
                pltpu.async_copy(h_hbm.at[src_v.at[j + 1]], rows_b, gsem_b)
                pltpu.async_copy(e_hbm.at[pl.ds(e_off(si, j + 1), C)],
                                 e_b, esem_b)
            pltpu.make_async_copy(h_hbm.at[src_v.at[j]], rows_a,
                                  gsem_a).wait()
            pltpu.make_async_copy(e_hbm.at[pl.ds(e_off(si, j), C)], e_a,
                                  esem_a).wait()
            compute_relu(rows_a, e_a)
            pltpu.async_copy(rows_a, agg_sh.at[dst_v.at[j]], ssem_a,
                             add=True)

        def chunk(j, carry1):
            @pl.when(j % 2 == 0)
            def _():
                step(j, rows0, e_v0, gsem0, ssem0, esem0,
                     rows1, e_v1, gsem1, ssem1, esem1)

            @pl.when(j % 2 == 1)
            def _():
                step(j, rows1, e_v1, gsem1, ssem1, esem1,
                     rows0, e_v0, gsem0, ssem0, esem0)
            return carry1
        lax.fori_loop(0, SUPC, chunk, None)
        lastp = (SUPC - 1) % 2
        pltpu.make_async_copy(
            rows0 if lastp == 0 else rows1,
            agg_sh.at[dst_v.at[SUPC - 1]],
            ssem0 if lastp == 0 else ssem1).wait()
        return carry
    lax.fori_loop(0, NSUP, sup, None)

    plsc.subcore_barrier()
    for kk in range(-(-NWB // NS)):
        jwb = s + kk * NS

        @pl.when(jwb < NWB)
        def _(jwb=jwb):
            r0 = pl.multiple_of(jwb * ZROWS, 8)
            pltpu.sync_copy(agg_sh.at[pl.ds(r0, ZROWS)], rows0)
            pltpu.sync_copy(rows0, out_hbm.at[c, pl.ds(r0, ZROWS)])


@functools.cache
def _make_edge_kernel():
    return functools.partial(
        pl.kernel,
        mesh=plsc.VectorSubcoreMesh(core_axis_name="c",
                                    subcore_axis_name="s"),
        out_type=jax.ShapeDtypeStruct((NC, N, D), jnp.float32),
        scratch_types=[
            pltpu.VMEM((SUPC, C), jnp.int32),
            pltpu.VMEM((SUPC, C), jnp.int32),
            pltpu.VMEM((C, D), jnp.float32),
            pltpu.VMEM((C, D), jnp.float32),
            pltpu.VMEM((C, D), jnp.float32),
            pltpu.VMEM((C, D), jnp.float32),
            pltpu.VMEM_SHARED((N, D), jnp.float32),
            pltpu.SemaphoreType.DMA,
            pltpu.SemaphoreType.DMA,
            pltpu.SemaphoreType.DMA,
            pltpu.SemaphoreType.DMA,
            pltpu.SemaphoreType.DMA,
            pltpu.SemaphoreType.DMA,
        ],
    )(_edge_body)


def _edge_kernel(h, e, src2d, dst2d):
    return _make_edge_kernel()(h, e, src2d, dst2d)


_BE = 2000


def _e_matmul(ea, We, be):
    def body(ea_ref, We_ref, be_ref, o_ref):
        o_ref[...] = (
            jnp.dot(ea_ref[...], We_ref[...],
                    preferred_element_type=jnp.float32) + be_ref[...]
        )
    return pl.pallas_call(
        body,
        grid=(E // _BE,),
        in_specs=[
            pl.BlockSpec((_BE, 37), lambda i: (i, 0)),
            pl.BlockSpec((37, D), lambda i: (0, 0)),
            pl.BlockSpec((1, D), lambda i: (0, 0)),
        ],
        out_specs=pl.BlockSpec((_BE, D), lambda i: (i, 0)),
        out_shape=jax.ShapeDtypeStruct((E, D), jnp.float32),
    )(ea, We, be)


_BN_B = 2000


def _node_matmul(h, p0, p1, Wn, bb):
    def body(h_ref, p0_ref, p1_ref, Wn_ref, bb_ref, t_ref, s_ref, ss_ref):
        i = pl.program_id(0)
        t = jnp.dot(h_ref[...] + p0_ref[...] + p1_ref[...], Wn_ref[...],
                    preferred_element_type=jnp.float32) + bb_ref[...]
        t_ref[...] = t

        @pl.when(i == 0)
        def _():
            s_ref[...] = jnp.zeros_like(s_ref)
            ss_ref[...] = jnp.zeros_like(ss_ref)
        s_ref[...] += jnp.sum(t, axis=0, keepdims=True)
        ss_ref[...] += jnp.sum(t * t, axis=0, keepdims=True)

    return pl.pallas_call(
        body,
        grid=(N // _BN_B,),
        in_specs=[
            pl.BlockSpec((_BN_B, D), lambda i: (i, 0)),
            pl.BlockSpec((_BN_B, D), lambda i: (i, 0)),
            pl.BlockSpec((_BN_B, D), lambda i: (i, 0)),
            pl.BlockSpec((D, D), lambda i: (0, 0)),
            pl.BlockSpec((1, D), lambda i: (0, 0)),
        ],
        out_specs=[
            pl.BlockSpec((_BN_B, D), lambda i: (i, 0)),
            pl.BlockSpec((1, D), lambda i: (0, 0)),
            pl.BlockSpec((1, D), lambda i: (0, 0)),
        ],
        out_shape=[
            jax.ShapeDtypeStruct((N, D), jnp.float32),
            jax.ShapeDtypeStruct((1, D), jnp.float32),
            jax.ShapeDtypeStruct((1, D), jnp.float32),
        ],
    )(h, p0, p1, Wn, bb)


def _bn_relu(t, ssum, ssq, g, bt):
    def body(t_ref, s_ref, ss_ref, g_ref, bt_ref, o_ref):
        mu = s_ref[...] / N
        var = ss_ref[...] / N - mu * mu
        o_ref[...] = jnp.maximum(
            (t_ref[...] - mu) * lax.rsqrt(var + 1e-5) * g_ref[...]
            + bt_ref[...], 0.0)

    return pl.pallas_call(
        body,
        grid=(N // _BN_B,),
        in_specs=[
            pl.BlockSpec((_BN_B, D), lambda i: (i, 0)),
            pl.BlockSpec((1, D), lambda i: (0, 0)),
            pl.BlockSpec((1, D), lambda i: (0, 0)),
            pl.BlockSpec((1, D), lambda i: (0, 0)),
            pl.BlockSpec((1, D), lambda i: (0, 0)),
        ],
        out_specs=pl.BlockSpec((_BN_B, D), lambda i: (i, 0)),
        out_shape=jax.ShapeDtypeStruct((N, D), jnp.float32),
    )(t, ssum, ssq, g, bt)


def _head(hp, bp, hg0, hbt0, hW1, hb1, hg1, hbt1, w2, b2):
    def bn(v, gg, bb_):
        mu = jnp.mean(v, axis=0, keepdims=True)
        var = jnp.mean((v - mu) ** 2, axis=0, keepdims=True)
        return (v - mu) * lax.rsqrt(var + 1e-5) * gg + bb_

    def body(h_ref, b_ref, hg0_r, hbt0_r, hW1_r, hb1_r, hg1_r, hbt1_r,
             w2_r, b2_r, o_ref):
        gid = lax.broadcasted_iota(jnp.int32, (G, NPAD), 0)
        m = (gid == b_ref[...]).astype(jnp.float32)
        sums = jnp.dot(m, h_ref[...], preferred_element_type=jnp.float32)
        cnt = jnp.sum(m, axis=1, keepdims=True)
        gp = sums / jnp.maximum(cnt, 1.0)
        o = bn(gp, hg0_r[...], hbt0_r[...])
        o = jnp.maximum(
            jnp.dot(o, hW1_r[...], preferred_element_type=jnp.float32)
            + hb1_r[...], 0.0)
        o = bn(o, hg1_r[...], hbt1_r[...])
        o_ref[...] = jnp.sum(o * w2_r[...], axis=1, keepdims=True) + b2_r[...]

    return pl.pallas_call(
        body,
        out_shape=jax.ShapeDtypeStruct((G, 1), jnp.float32),
    )(hp, bp, hg0, hbt0, hW1, hb1, hg1, hbt1, w2, b2)


def kernel(x, edge_attr, We0, be0, Wn0, bb0, g0, bt0, We1, be1, Wn1, bb1,
           g1, bt1, We2, be2, Wn2, bb2, g2, bt2, hg0, hbt0, hW1, hb1, hg1,
           hbt1, hW2, hb2, edge_index, batch):
    x0 = jax.nn.one_hot(x[:, 0].astype(jnp.int32), 119, dtype=jnp.float32)
    h = jnp.concatenate([x0, x[:, 1:]], axis=1)
    ea0 = jax.nn.one_hot(edge_attr[:, 0].astype(jnp.int32), 22,
                         dtype=jnp.float32)
    ea = jnp.concatenate([ea0, edge_attr[:, 1:]], axis=1)
    src2d = edge_index[0].reshape(NW, NSUP, SUPC, C)
    dst2d = edge_index[1].reshape(NW, NSUP, SUPC, C)

    layers = [(We0, be0, Wn0, bb0, g0, bt0),
              (We1, be1, Wn1, bb1, g1, bt1),
              (We2, be2, Wn2, bb2, g2, bt2)]
    e = _e_matmul(ea, We0, be0.reshape(1, D))
    for li, (We, be, Wn, bb, g, bt) in enumerate(layers):
        parts = _edge_kernel(h, e, src2d, dst2d)
        if li + 1 < len(layers):
            Wen, ben = layers[li + 1][0], layers[li + 1][1]
            e = _e_matmul(ea, Wen, ben.reshape(1, D))
        t, ssum, ssq = _node_matmul(h, parts[0], parts[1], Wn,
                                    bb.reshape(1, D))
        h = _bn_relu(t, ssum, ssq, g.reshape(1, D), bt.reshape(1, D))

    hp = jnp.pad(h, ((0, NPAD - N), (0, 0)))
    bp = jnp.pad(batch, (0, NPAD - N), constant_values=G).reshape(1, NPAD)
    return _head(hp, bp, hg0.reshape(1, D), hbt0.reshape(1, D), hW1,
                 hb1.reshape(1, D), hg1.reshape(1, D), hbt1.reshape(1, D),
                 hW2.reshape(1, D), hb2.reshape(1, 1))
